# Initial kernel scaffold; baseline (speedup 1.0000x reference)
import jax
import jax.numpy as jnp
from jax.experimental import pallas as pl


def kernel(x, edge_index, edge_attr, u, batch, edge_w1, edge_b1, edge_w2, edge_b2, node1_w1, node1_b1, node1_w2, node1_b2, node2_w1, node2_b1, node2_w2, node2_b2, glob_w1, glob_b1, glob_w2, glob_b2):
    raise NotImplementedError("write your pallas kernel here")


# speedup vs baseline: 2.0546x; 2.0546x over previous
import functools

import jax
import jax.numpy as jnp
from jax import lax
from jax.experimental import pallas as pl
from jax.experimental.pallas import tpu as pltpu
from jax.experimental.pallas import tpu_sc as plsc

_N = 10000
_E = 160000
_F = 128
_FE = 16
_FG = 16
_H = 256
_G = 64

_NP = 10240
_BN = 1000
_BE = 1000
_NC = 2
_NS = 16
_NW = _NC * _NS

_f32 = jnp.float32


def _node_pre_body(x_ref, oh_ref, u_ref, wcat_ref, wed_ref, eb1_ref,
                   n1b1_ref, t1_ref, xb_ref):
    x = x_ref[...]
    xc = jnp.dot(x, wcat_ref[...], preferred_element_type=_f32)
    ug2 = jnp.dot(u_ref[...], wed_ref[...], preferred_element_type=_f32) + eb1_ref[...]
    t1a = xc[:, :_H] + jnp.dot(oh_ref[...], ug2, preferred_element_type=_f32)
    t1b = xc[:, _H:2 * _H] + n1b1_ref[...]
    t1_ref[...] = jnp.concatenate([t1a, t1b], axis=1)
    xb_ref[...] = xc[:, 2 * _H:]


def _node_pre(x, oh, u, wcat, wed, eb1, n1b1):
    nb = _N // _BN
    return pl.pallas_call(
        _node_pre_body,
        grid=(nb,),
        in_specs=[
            pl.BlockSpec((_BN, _F), lambda i: (i, 0)),
            pl.BlockSpec((_BN, _G), lambda i: (i, 0)),
            pl.BlockSpec((_G, _FG), lambda i: (0, 0)),
            pl.BlockSpec((_F, 3 * _H), lambda i: (0, 0)),
            pl.BlockSpec((_FG, _H), lambda i: (0, 0)),
            pl.BlockSpec((1, _H), lambda i: (0, 0)),
            pl.BlockSpec((1, _H), lambda i: (0, 0)),
        ],
        out_specs=[
            pl.BlockSpec((_BN, 2 * _H), lambda i: (i, 0)),
            pl.BlockSpec((_BN, _H), lambda i: (i, 0)),
        ],
        out_shape=[
            jax.ShapeDtypeStruct((_N, 2 * _H), _f32),
            jax.ShapeDtypeStruct((_N, _H), _f32),
        ],
    )(x, oh, u, wcat, wed, eb1, n1b1)


def _gather_body(t1_h, xb_h, row_h, col_h, o1_h, o2_h,
                 i1_v, i2_v, r1_v, r2_v, s1, s2):
    K = 40
    per_w = _E // _NW
    wid = lax.axis_index("s") * _NC + lax.axis_index("c")
    base0 = wid * per_w

    def body(i, carry):
        b = base0 + i * K
        pltpu.sync_copy(row_h.at[pl.ds(b, K)], i1_v)
        pltpu.sync_copy(col_h.at[pl.ds(b, K)], i2_v)
        c1 = pltpu.async_copy(t1_h.at[i1_v], r1_v, s1)
        c2 = pltpu.async_copy(xb_h.at[i2_v], r2_v, s2)
        c1.wait()
        c2.wait()
        pltpu.sync_copy(r1_v, o1_h.at[pl.ds(b, K)])
        pltpu.sync_copy(r2_v, o2_h.at[pl.ds(b, K)])
        return carry

    lax.fori_loop(0, per_w // K, body, 0)


def _sc_gather(t1, xb, row, col):
    K = 40
    f = pl.kernel(
        _gather_body,
        out_type=[
            jax.ShapeDtypeStruct((_E, 2 * _H), _f32),
            jax.ShapeDtypeStruct((_E, _H), _f32),
        ],
        mesh=plsc.VectorSubcoreMesh(core_axis_name="c", subcore_axis_name="s"),
        scratch_types=[
            pltpu.VMEM((K,), jnp.int32),
            pltpu.VMEM((K,), jnp.int32),
            pltpu.VMEM((K, 2 * _H), _f32),
            pltpu.VMEM((K, _H), _f32),
            pltpu.SemaphoreType.DMA,
            pltpu.SemaphoreType.DMA,
        ],
    )
    return f(t1, xb, row, col)


def _edge_body(g1_ref, g2_ref, ea_ref, wec_ref, ew2_ref, eb2_ref, wn1b_ref,
               ean_ref, h1n_ref):
    g1 = g1_ref[...]
    base = g1[:, :_H] + g2_ref[...]
    h1e = jnp.maximum(base + jnp.dot(ea_ref[...], wec_ref[...],
                                     preferred_element_type=_f32), 0.0)
    ean = jnp.dot(h1e, ew2_ref[...], preferred_element_type=_f32) + eb2_ref[...]
    h1n = jnp.maximum(g1[:, _H:] + jnp.dot(ean, wn1b_ref[...],
                                           preferred_element_type=_f32), 0.0)
    ean_ref[...] = ean
    h1n_ref[0] = h1n[:, :_F]
    h1n_ref[1] = h1n[:, _F:]


def _edge_dense(g1, g2, ea, wec, ew2, eb2, wn1b):
    nb = _E // _BE
    return pl.pallas_call(
        _edge_body,
        grid=(nb,),
        in_specs=[
            pl.BlockSpec((_BE, 2 * _H), lambda i: (i, 0)),
            pl.BlockSpec((_BE, _H), lambda i: (i, 0)),
            pl.BlockSpec((_BE, _FE), lambda i: (i, 0)),
            pl.BlockSpec((_FE, _H), lambda i: (0, 0)),
            pl.BlockSpec((_H, _FE), lambda i: (0, 0)),
            pl.BlockSpec((1, _FE), lambda i: (0, 0)),
            pl.BlockSpec((_FE, _H), lambda i: (0, 0)),
        ],
        out_specs=[
            pl.BlockSpec((_BE, _FE), lambda i: (i, 0)),
            pl.BlockSpec((2, _BE, _F), lambda i: (0, i, 0)),
        ],
        out_shape=[
            jax.ShapeDtypeStruct((_E, _FE), _f32),
            jax.ShapeDtypeStruct((2, _E, _F), _f32),
        ],
    )(g1, g2, ea, wec, ew2, eb2, wn1b)


def _scatter_body(h1n_h, col_h, zer_h, out_h, idx_v, buf_v, acc_sh):
    K = 80
    rows_per_tile = _NP // _NS
    per_tile = _E // _NS
    c = lax.axis_index("c")
    s = lax.axis_index("s")

    pltpu.sync_copy(zer_h.at[pl.ds(s * rows_per_tile, rows_per_tile)],
                    acc_sh.at[pl.ds(s * rows_per_tile, rows_per_tile)])
    plsc.subcore_barrier()

    def body(i, carry):
        b = s * per_tile + i * K
        pltpu.sync_copy(col_h.at[pl.ds(b, K)], idx_v)
        pltpu.sync_copy(h1n_h.at[c, pl.ds(b, K)], buf_v)
        pltpu.sync_copy(buf_v, acc_sh.at[idx_v], add=True)
        return carry

    lax.fori_loop(0, per_tile // K, body, 0)
    plsc.subcore_barrier()
    pltpu.sync_copy(acc_sh.at[pl.ds(s * rows_per_tile, rows_per_tile)],
                    out_h.at[c, pl.ds(s * rows_per_tile, rows_per_tile)])


def _sc_scatter(h1n, col, zer):
    K = 80
    f = pl.kernel(
        _scatter_body,
        out_type=jax.ShapeDtypeStruct((2, _NP, _F), _f32),
        mesh=plsc.VectorSubcoreMesh(core_axis_name="c", subcore_axis_name="s"),
        scratch_types=[
            pltpu.VMEM((K,), jnp.int32),
            pltpu.VMEM((K, _F), _f32),
            pltpu.VMEM_SHARED((_NP, _F), _f32),
        ],
    )
    return f(h1n, col, zer)


def _deg_body(col_h, zer_h, ones_h, out_h, idx_v, ones_v, acc_sh):
    K = 80
    rows_per_tile = _NP // _NS
    per_tile = _E // _NS
    c = lax.axis_index("c")
    s = lax.axis_index("s")

    @pl.when(c == 0)
    def _():
        pltpu.sync_copy(ones_h, ones_v)
        pltpu.sync_copy(zer_h.at[pl.ds(s * rows_per_tile, rows_per_tile)],
                        acc_sh.at[pl.ds(s * rows_per_tile, rows_per_tile)])
        plsc.subcore_barrier()

        def body(i, carry):
            b = s * per_tile + i * K
            pltpu.sync_copy(col_h.at[pl.ds(b, K)], idx_v)
            pltpu.sync_copy(ones_v, acc_sh.at[idx_v], add=True)
            return carry

        lax.fori_loop(0, per_tile // K, body, 0)
        plsc.subcore_barrier()
        pltpu.sync_copy(acc_sh.at[pl.ds(s * rows_per_tile, rows_per_tile)],
                        out_h.at[pl.ds(s * rows_per_tile, rows_per_tile)])


def _sc_degree(col, zer):
    K = 80
    ones = jnp.ones((K, _F), _f32)
    f = pl.kernel(
        _deg_body,
        out_type=jax.ShapeDtypeStruct((_NP, _F), _f32),
        mesh=plsc.VectorSubcoreMesh(core_axis_name="c", subcore_axis_name="s"),
        scratch_types=[
            pltpu.VMEM((K,), jnp.int32),
            pltpu.VMEM((K, _F), _f32),
            pltpu.VMEM_SHARED((_NP, _F), _f32),
        ],
    )
    return f(col, zer, ones)


def _node_post_body(x_ref, agg_ref, deg_ref, oh_ref, u_ref,
                    n1w2_ref, n1b2_ref, wn2a_ref, wn2b_ref, wn2c_ref,
                    n2b1_ref, n2w2_ref, n2b2_ref, xn_ref):
    deg = deg_ref[:, :1]
    aggf = jnp.concatenate([agg_ref[0], agg_ref[1]], axis=1)
    aggm = aggf / jnp.maximum(deg, 1.0)
    aggv = (jnp.dot(aggm, n1w2_ref[...], preferred_element_type=_f32)
            + n1b2_ref[...] * jnp.minimum(deg, 1.0))
    ub = jnp.dot(oh_ref[...],
                 jnp.dot(u_ref[...], wn2c_ref[...], preferred_element_type=_f32),
                 preferred_element_type=_f32)
    h2 = jnp.maximum(
        jnp.dot(x_ref[...], wn2a_ref[...], preferred_element_type=_f32)
        + jnp.dot(aggv, wn2b_ref[...], preferred_element_type=_f32)
        + ub + n2b1_ref[...], 0.0)
    xn_ref[...] = jnp.dot(h2, n2w2_ref[...], preferred_element_type=_f32) + n2b2_ref[...]


def _node_post(x, agg, deg16, oh, u, n1w2, n1b2, wn2a, wn2b, wn2c, n2b1, n2w2, n2b2):
    nb = _N // _BN
    return pl.pallas_call(
        _node_post_body,
        grid=(nb,),
        in_specs=[
            pl.BlockSpec((_BN, _F), lambda i: (i, 0)),
            pl.BlockSpec((2, _BN, _F), lambda i: (0, i, 0)),
            pl.BlockSpec((_BN, _F), lambda i: (i, 0)),
            pl.BlockSpec((_BN, _G), lambda i: (i, 0)),
            pl.BlockSpec((_G, _FG), lambda i: (0, 0)),
            pl.BlockSpec((_H, _H), lambda i: (0, 0)),
            pl.BlockSpec((1, _H), lambda i: (0, 0)),
            pl.BlockSpec((_F, _H), lambda i: (0, 0)),
            pl.BlockSpec((_H, _H), lambda i: (0, 0)),
            pl.BlockSpec((_FG, _H), lambda i: (0, 0)),
            pl.BlockSpec((1, _H), lambda i: (0, 0)),
            pl.BlockSpec((_H, _F), lambda i: (0, 0)),
            pl.BlockSpec((1, _F), lambda i: (0, 0)),
        ],
        out_specs=pl.BlockSpec((_BN, _F), lambda i: (i, 0)),
        out_shape=jax.ShapeDtypeStruct((_N, _F), _f32),
    )(x, agg, deg16, oh, u, n1w2, n1b2, wn2a, wn2b, wn2c, n2b1, n2w2, n2b2)


def _glob_body(xn_ref, oh_ref, u_ref, wga_ref, wgb_ref, gb1_ref, gw2_ref,
               gb2_ref, un_ref):
    oh = oh_ref[...]
    xs = lax.dot_general(oh, xn_ref[...], (((0,), (0,)), ((), ())),
                         preferred_element_type=_f32)
    cnt = jnp.sum(oh, axis=0)[:, None]
    xm = xs / jnp.maximum(cnt, 1.0)
    h = jnp.maximum(
        jnp.dot(u_ref[...], wga_ref[...], preferred_element_type=_f32)
        + jnp.dot(xm, wgb_ref[...], preferred_element_type=_f32)
        + gb1_ref[...], 0.0)
    un_ref[...] = jnp.dot(h, gw2_ref[...], preferred_element_type=_f32) + gb2_ref[...]


def _glob_update(xn, oh, u, wga, wgb, gb1, gw2, gb2):
    return pl.pallas_call(
        _glob_body,
        out_shape=jax.ShapeDtypeStruct((_G, _FG), _f32),
    )(xn, oh, u, wga, wgb, gb1, gw2, gb2)


def kernel(x, edge_index, edge_attr, u, batch,
           edge_w1, edge_b1, edge_w2, edge_b2,
           node1_w1, node1_b1, node1_w2, node1_b2,
           node2_w1, node2_b1, node2_w2, node2_b2,
           glob_w1, glob_b1, glob_w2, glob_b2):
    row = edge_index[0].astype(jnp.int32)
    col = edge_index[1].astype(jnp.int32)

    oh = (batch[:, None] == jnp.arange(_G, dtype=batch.dtype)[None, :]).astype(_f32)

    wea = edge_w1[:_F]
    web = edge_w1[_F:2 * _F]
    wec = edge_w1[2 * _F:2 * _F + _FE]
    wed = edge_w1[2 * _F + _FE:]
    wn1a = node1_w1[:_F]
    wn1b = node1_w1[_F:]
    wn2a = node2_w1[:_F]
    wn2b = node2_w1[_F:_F + _H]
    wn2c = node2_w1[_F + _H:]
    wga = glob_w1[:_FG]
    wgb = glob_w1[_FG:]
    wcat = jnp.concatenate([wea, wn1a, web], axis=1)

    eb1 = edge_b1[None, :]
    n1b1 = node1_b1[None, :]
    eb2 = edge_b2[None, :]
    n1b2 = node1_b2[None, :]
    n2b1 = node2_b1[None, :]
    n2b2 = node2_b2[None, :]
    gb1 = glob_b1[None, :]
    gb2 = glob_b2[None, :]

    zer = jnp.zeros((_NP, _F), _f32)

    deg16 = _sc_degree(col, zer)

    for _ in range(3):
        t1, xb = _node_pre(x, oh, u, wcat, wed, eb1, n1b1)
        g1, g2 = _sc_gather(t1, xb, row, col)
        ean, h1n = _edge_dense(g1, g2, edge_attr, wec, edge_w2, eb2, wn1b)
        agg = _sc_scatter(h1n, col, zer)
        x_new = _node_post(x, agg, deg16, oh, u, node1_w2, n1b2,
                           wn2a, wn2b, wn2c, n2b1, node2_w2, n2b2)
        u = _glob_update(x_new, oh, u, wga, wgb, gb1, gw2=glob_w2, gb2=gb2)
        x = x_new
        edge_attr = ean

    return (x, edge_attr, u)
